# baseline (speedup 1.0000x reference)
import jax
import jax.numpy as jnp
from jax import lax
from jax.experimental import pallas as pl
from jax.experimental.pallas import tpu as pltpu

N_NODES = 10000
N_EDGES = 320000
CH = 128

E = 4000
NB = N_EDGES // E
G = 16
NG = E // G
W = 128
ACCN = 10240
NEG = -3.0e38


def _shift_up_f(a, k, fill):
    return jnp.concatenate(
        [a[k:], jnp.full((k,) + a.shape[1:], fill, a.dtype)], axis=0)


def _body(vid_ref, x_ref, w1t_ref, b1_ref, wet_ref, wvt_ref, out_ref, acc_ref):
    p = pl.program_id(0)
    b = pl.program_id(1)
    vcol = vid_ref[0]
    v_first = vcol[0, 0]
    v_last = vcol[E - 1, 0]
    base = (v_first // 8) * 8
    nchunks = (v_last - base) // W + 1
    vbb = jnp.broadcast_to(vcol, (E, CH))
    lane = lax.broadcasted_iota(jnp.int32, (E, W), 1)

    @pl.when((p == 0) & (b == 0))
    def _init():
        acc_ref[...] = jnp.full((ACCN, CH), NEG, jnp.float32)

    @pl.when(p == 0)
    def _phase0():
        xb = x_ref[...].astype(jnp.bfloat16)
        z = jnp.dot(xb, w1t_ref[...], preferred_element_type=jnp.float32)
        z = z + b1_ref[...]
        s = z.astype(jnp.bfloat16)
        vb16 = vbb.astype(jnp.int16)
        k = 1
        while k < G:
            s = jnp.where(vb16 == _shift_up_f(vb16, k, -1),
                          jnp.maximum(s, _shift_up_f(s, k, NEG)), s)
            k *= 2
        heads = s.reshape(NG, G, CH)[:, 0, :]
        vheads = vb16.reshape(NG, G, CH)[:, 0, :]
        k = 1
        while k < NG:
            heads = jnp.where(vheads == _shift_up_f(vheads, k, -1),
                              jnp.maximum(heads, _shift_up_f(heads, k, NEG)),
                              heads)
            k *= 2
        tnext = _shift_up_f(heads, 1, NEG)
        vnext = _shift_up_f(vheads, 1, -1)
        trep = jnp.broadcast_to(tnext[:, None, :], (NG, G, CH)).reshape(E, CH)
        vrep = jnp.broadcast_to(vnext[:, None, :], (NG, G, CH)).reshape(E, CH)
        s = jnp.where(vb16 == vrep, jnp.maximum(s, trep), s)
        prevv = jnp.concatenate(
            [jnp.full((1, CH), -1, jnp.int16), vb16[:E - 1]], axis=0)
        firstb = (vb16 != prevv)
        lane16 = lane.astype(jnp.int16)
        one16 = jnp.bfloat16(1.0)
        zero16 = jnp.bfloat16(0.0)
        ones_ec = jnp.ones((E, CH), jnp.bfloat16)

        def chunk(j, _):
            start16 = (base + j * W).astype(jnp.int16)
            rel = vb16 - start16
            sel = jnp.where(firstb & (rel == lane16), one16, zero16)
            upd = lax.dot_general(sel, s, (((0,), (0,)), ((), ())),
                                  preferred_element_type=jnp.float32)
            cnt = lax.dot_general(sel, ones_ec, (((0,), (0,)), ((), ())),
                                  preferred_element_type=jnp.float32)
            cur = acc_ref[pl.ds(base + j * W, W), :]
            acc_ref[pl.ds(base + j * W, W), :] = jnp.where(
                cnt > 0.0, jnp.maximum(cur, upd), cur)
            return 0

        lax.fori_loop(0, nchunks, chunk, 0)

    @pl.when((p == 1) & (b == 0))
    def _apply_wv():
        def step(i, _):
            sl = acc_ref[pl.ds(i * 512, 512), :]
            sl = jnp.where(sl == NEG, 0.0, sl)
            acc_ref[pl.ds(i * 512, 512), :] = jnp.dot(
                sl, wvt_ref[...], preferred_element_type=jnp.float32)
            return 0
        lax.fori_loop(0, ACCN // 512, step, 0)

    @pl.when(p == 1)
    def _phase1():
        xb = x_ref[...].astype(jnp.bfloat16)
        ze = jnp.dot(xb, wet_ref[...], preferred_element_type=jnp.float32)
        vb16 = vbb.astype(jnp.int16)
        lane16 = lane.astype(jnp.int16)

        def chunk(j, g):
            rel = vb16 - (base + j * W).astype(jnp.int16)
            sel = jnp.where(rel == lane16,
                            jnp.bfloat16(1.0), jnp.bfloat16(0.0))
            zw = acc_ref[pl.ds(base + j * W, W), :].astype(jnp.bfloat16)
            return g + jnp.dot(sel, zw, preferred_element_type=jnp.float32)

        g = lax.fori_loop(0, nchunks, chunk, jnp.zeros((E, CH), jnp.float32))
        out_ref[...] = ze + g


def kernel(x, vertex_id, W1, b1, We, Wv):
    vid3 = vertex_id.astype(jnp.int32).reshape(NB, E, 1)
    grid = (2, NB)
    return pl.pallas_call(
        _body,
        grid=grid,
        in_specs=[
            pl.BlockSpec((1, E, 1), lambda p, b: (b, 0, 0)),
            pl.BlockSpec((E, CH), lambda p, b: (b, 0)),
            pl.BlockSpec((CH, CH), lambda p, b: (0, 0)),
            pl.BlockSpec((1, CH), lambda p, b: (0, 0)),
            pl.BlockSpec((CH, CH), lambda p, b: (0, 0)),
            pl.BlockSpec((CH, CH), lambda p, b: (0, 0)),
        ],
        out_specs=pl.BlockSpec(
            (E, CH), lambda p, b: (jnp.where(p == 1, b, 0), 0)),
        out_shape=jax.ShapeDtypeStruct((N_EDGES, CH), jnp.float32),
        scratch_shapes=[pltpu.VMEM((ACCN, CH), jnp.float32)],
        compiler_params=pltpu.CompilerParams(
            dimension_semantics=("arbitrary", "arbitrary")),
    )(vid3, x, W1.T.astype(jnp.bfloat16), b1.reshape(1, CH),
      We.T.astype(jnp.bfloat16), Wv.T)


# speedup vs baseline: 1.5175x; 1.5175x over previous
import functools
import jax
import jax.numpy as jnp
from jax import lax
from jax.experimental import pallas as pl
from jax.experimental.pallas import tpu as pltpu
from jax.experimental.pallas import tpu_sc as plsc

N_NODES = 10000
N_EDGES = 320000
CH = 128

EZ = 2560
NBZ = N_EDGES // EZ


def _zbody(x_ref, w1t_ref, b1_ref, z_ref):
    z_ref[...] = jnp.dot(x_ref[...].astype(jnp.bfloat16), w1t_ref[...],
                         preferred_element_type=jnp.float32) + b1_ref[...]


def _make_z(x, w1t, b1):
    return pl.pallas_call(
        _zbody,
        grid=(NBZ,),
        in_specs=[
            pl.BlockSpec((EZ, CH), lambda b: (b, 0)),
            pl.BlockSpec((CH, CH), lambda b: (0, 0)),
            pl.BlockSpec((1, CH), lambda b: (0, 0)),
        ],
        out_specs=pl.BlockSpec((EZ, CH), lambda b: (b, 0)),
        out_shape=jax.ShapeDtypeStruct((N_EDGES, CH), jnp.float32),
    )(x, w1t, b1)


NW = 32
EW = N_EDGES // NW
C = 400
NCHUNK = EW // C
NGRP = C // 16
K = 16
OUTN = N_NODES + NW + 16
NEG = -3.0e38


def _sc_segmax(z, vid):
    mesh = plsc.VectorSubcoreMesh(core_axis_name="c", subcore_axis_name="s",
                                  num_cores=2, num_subcores=16)

    @functools.partial(
        pl.kernel,
        mesh=mesh,
        out_type=jax.ShapeDtypeStruct((OUTN, CH), jnp.float32),
        scratch_types=[
            pltpu.VMEM((C, CH), jnp.float32),
            pltpu.VMEM((C, CH), jnp.float32),
            pltpu.VMEM((C,), jnp.int32),
            pltpu.VMEM((C,), jnp.int32),
            pltpu.VMEM((K, CH), jnp.float32),
            pltpu.VMEM((K,), jnp.int32),
            pltpu.SemaphoreType.DMA,
            pltpu.SemaphoreType.DMA,
            pltpu.SemaphoreType.DMA,
        ],
        compiler_params=pltpu.CompilerParams(needs_layout_passes=False),
    )
    def k(z_hbm, vid_hbm, out_hbm, zbuf0, zbuf1, vbuf0, vbuf1, rbuf, ribuf,
          sem0, sem1, wsem):
        wid = lax.axis_index("s") * 2 + lax.axis_index("c")
        dummy = N_NODES + NW + lax.rem(wid, 16)
        dummyvec = jnp.full((K,), 1, jnp.int32) * dummy
        lane16 = lax.iota(jnp.int32, 16)
        base = wid * EW

        def start_fetch(ci, zb, vb, sem):
            pltpu.async_copy(z_hbm.at[pl.ds(base + ci * C, C)], zb, sem)
            pltpu.async_copy(vid_hbm.at[pl.ds(base + ci * C, C)], vb, sem)

        def wait_fetch(zb, vb, sem):
            pltpu.make_async_copy(z_hbm.at[pl.ds(0, C)], zb, sem).wait()
            pltpu.make_async_copy(vid_hbm.at[pl.ds(0, C)], vb, sem).wait()

        def scatter_flush(idxv):
            ribuf[...] = idxv
            pltpu.async_copy(rbuf, out_hbm.at[ribuf], wsem)
            pltpu.make_async_copy(rbuf, out_hbm.at[ribuf], wsem).wait()

        def process(zb, vb, carry):
            def grp_body(g, gc):
                vgrp = vb[pl.ds(g * 16, 16)]
                carry2 = gc
                for j in range(16):
                    cur_vid, pos, idxv = carry2[0], carry2[1], carry2[2]
                    acc = carry2[3:]
                    v = vgrp[j]
                    r = g * 16 + j
                    row = [zb[r, pl.ds(q * 16, 16)] for q in range(8)]
                    same = v == cur_vid
                    close = jnp.logical_not(same) & (cur_vid >= 0)

                    @pl.when(close)
                    def _close():
                        rowidx = jnp.full((16,), 1, jnp.int32) * pos
                        for q in range(8):
                            plsc.store_scatter(
                                rbuf, [rowidx, q * 16 + lane16], acc[q])

                    idxv2 = jnp.where(close & (lane16 == pos),
                                      jnp.full((K,), 1, jnp.int32) * cur_vid,
                                      idxv)
                    pos2 = jnp.where(close, pos + 1, pos)
                    fire = pos2 == K

                    @pl.when(fire)
                    def _fire():
                        scatter_flush(idxv2)

                    idxv3 = jnp.where(fire, dummyvec, idxv2)
                    pos3 = jnp.where(fire, 0, pos2)
                    newacc = [
                        jnp.where(same, jnp.maximum(acc[q], row[q]), row[q])
                        for q in range(8)
                    ]
                    carry2 = (v, pos3, idxv3) + tuple(newacc)
                return carry2

            return lax.fori_loop(0, NGRP, grp_body, carry)

        init = (jnp.int32(-1), jnp.int32(0), dummyvec) + tuple(
            jnp.full((16,), NEG, jnp.float32) for _ in range(8))

        start_fetch(0, zbuf0, vbuf0, sem0)

        def pair_body(pi, carry):
            wait_fetch(zbuf0, vbuf0, sem0)
            start_fetch(2 * pi + 1, zbuf1, vbuf1, sem1)
            carry = process(zbuf0, vbuf0, carry)
            wait_fetch(zbuf1, vbuf1, sem1)
            start_fetch(2 * pi + 2, zbuf0, vbuf0, sem0)
            carry = process(zbuf1, vbuf1, carry)
            return carry

        fin = lax.fori_loop(0, (NCHUNK - 1) // 2, pair_body, init)
        wait_fetch(zbuf0, vbuf0, sem0)
        fin = process(zbuf0, vbuf0, fin)

        scatter_flush(fin[2])
        zeroidx = jnp.full((16,), 0, jnp.int32)
        for q in range(8):
            plsc.store_scatter(rbuf, [zeroidx, q * 16 + lane16], fin[3 + q])
        scatter_flush(jnp.where(lane16 == 0, N_NODES + wid, dummy))

    return k(z, vid)


E = 2560
NB = N_EDGES // E
W = 128
ACCN = 10240


def _fbody(vid_ref, pid_ref, rep_ref, x_ref, zm_ref, wet_ref, wvt_ref,
           out_ref, acc_ref):
    b = pl.program_id(0)
    vcol = vid_ref[0]
    v_first = vcol[0, 0]
    v_last = vcol[E - 1, 0]
    base = (v_first // 8) * 8
    nchunks = (v_last - base) // W + 1
    vbb = jnp.broadcast_to(vcol, (E, CH))
    lane = lax.broadcasted_iota(jnp.int32, (E, W), 1)

    @pl.when(b == 0)
    def _prep():
        acc_ref[pl.ds(0, N_NODES), :] = zm_ref[pl.ds(0, N_NODES), :]
        acc_ref[pl.ds(N_NODES, ACCN - N_NODES), :] = jnp.full(
            (ACCN - N_NODES, CH), NEG, jnp.float32)

        def merge(w, _):
            pid = pid_ref[w]
            cur = acc_ref[pl.ds(pid, 1), :]
            part = zm_ref[pl.ds(N_NODES + w, 1), :]
            acc_ref[pl.ds(pid, 1), :] = jnp.where(
                rep_ref[w] == 1, part, jnp.maximum(cur, part))
            return 0
        lax.fori_loop(0, NW, merge, 0)

        def step(i, _):
            sl = acc_ref[pl.ds(i * 512, 512), :]
            sl = jnp.where(jnp.isfinite(sl) & (sl != NEG), sl, 0.0)
            res = jnp.dot(sl, wvt_ref[...], preferred_element_type=jnp.float32)
            acc_ref[pl.ds(i * 512, 512), :] = jnp.where(
                jnp.isfinite(res), res, 0.0)
            return 0
        lax.fori_loop(0, ACCN // 512, step, 0)

    ze = jnp.dot(x_ref[...].astype(jnp.bfloat16), wet_ref[...],
                 preferred_element_type=jnp.float32)
    vb16 = vbb.astype(jnp.int16)
    lane16 = lane.astype(jnp.int16)

    def chunk(j, g):
        rel = vb16 - (base + j * W).astype(jnp.int16)
        sel = jnp.where(rel == lane16, jnp.bfloat16(1.0), jnp.bfloat16(0.0))
        zw = acc_ref[pl.ds(base + j * W, W), :].astype(jnp.bfloat16)
        return g + jnp.dot(sel, zw, preferred_element_type=jnp.float32)

    g = lax.fori_loop(0, nchunks, chunk, jnp.zeros((E, CH), jnp.float32))
    out_ref[...] = ze + g


def _finish(vid3, pids, reps, x, zm, wet, wvt):
    return pl.pallas_call(
        _fbody,
        grid=(NB,),
        in_specs=[
            pl.BlockSpec((1, E, 1), lambda b: (b, 0, 0)),
            pl.BlockSpec(memory_space=pltpu.SMEM),
            pl.BlockSpec(memory_space=pltpu.SMEM),
            pl.BlockSpec((E, CH), lambda b: (b, 0)),
            pl.BlockSpec((OUTN, CH), lambda b: (0, 0)),
            pl.BlockSpec((CH, CH), lambda b: (0, 0)),
            pl.BlockSpec((CH, CH), lambda b: (0, 0)),
        ],
        out_specs=pl.BlockSpec((E, CH), lambda b: (b, 0)),
        out_shape=jax.ShapeDtypeStruct((N_EDGES, CH), jnp.float32),
        scratch_shapes=[pltpu.VMEM((ACCN, CH), jnp.float32)],
    )(vid3, pids, reps, x, zm, wet, wvt)


def kernel(x, vertex_id, W1, b1, We, Wv):
    vid = vertex_id.astype(jnp.int32)
    z = _make_z(x, W1.T.astype(jnp.bfloat16), b1.reshape(1, CH))
    zm = _sc_segmax(z, vid)
    vid3 = vid.reshape(NB, E, 1)
    pids = vid[EW - 1::EW]
    lastpos = jnp.searchsorted(vid, pids, side="right") - 1
    direct = (lastpos % EW) != (EW - 1)
    firsts = jnp.concatenate(
        [jnp.ones((1,), bool), pids[1:] != pids[:-1]])
    reps = ((~direct) & firsts).astype(jnp.int32)
    return _finish(vid3, pids, reps, x, zm,
                   We.T.astype(jnp.bfloat16), Wv.T)
